# Initial kernel scaffold; baseline (speedup 1.0000x reference)
import jax
import jax.numpy as jnp
from jax.experimental import pallas as pl


def kernel(s, r, o, t, E_w, R_w, T_w):
    raise NotImplementedError("write your pallas kernel here")


# speedup vs baseline: 3.4473x; 3.4473x over previous
import functools

import jax
import jax.numpy as jnp
from jax import lax
from jax.experimental import pallas as pl
from jax.experimental.pallas import tpu as pltpu
from jax.experimental.pallas import tpu_sc as plsc

DIM = 64
LANES = 16
NUM_WORKERS = 32
CHUNK = 128


def _sc_score(s_f, r_f, o_f, t_f, E_w, R_w, T_w):
    total = s_f.shape[0]
    per_w = total // NUM_WORKERS
    n_chunks = per_w // CHUNK

    mesh = plsc.VectorSubcoreMesh(core_axis_name="c", subcore_axis_name="s")

    @functools.partial(
        pl.kernel,
        mesh=mesh,
        out_type=jax.ShapeDtypeStruct((total,), jnp.float32),
        compiler_params=pltpu.CompilerParams(use_tc_tiling_on_sc=False),
        scratch_types=[
            pltpu.VMEM((CHUNK,), jnp.int32),
            pltpu.VMEM((CHUNK,), jnp.int32),
            pltpu.VMEM((CHUNK,), jnp.int32),
            pltpu.VMEM((CHUNK,), jnp.int32),
            pltpu.VMEM((CHUNK, DIM), jnp.float32),
            pltpu.VMEM((CHUNK, DIM), jnp.float32),
            pltpu.VMEM((CHUNK, DIM), jnp.float32),
            pltpu.VMEM((CHUNK, DIM), jnp.float32),
            pltpu.VMEM((CHUNK,), jnp.float32),
            pltpu.SemaphoreType.DMA,
        ],
    )
    def k(s_hbm, r_hbm, o_hbm, t_hbm, E_hbm, R_hbm, T_hbm, out_hbm,
          sidx, ridx, oidx, tidx, srows, rrows, orows, trows, outv, sem):
        wid = lax.axis_index("s") * 2 + lax.axis_index("c")
        base = wid * per_w

        def chunk_body(ci, _):
            off = base + ci * CHUNK
            pltpu.sync_copy(s_hbm.at[pl.ds(off, CHUNK)], sidx)
            pltpu.sync_copy(r_hbm.at[pl.ds(off, CHUNK)], ridx)
            pltpu.sync_copy(o_hbm.at[pl.ds(off, CHUNK)], oidx)
            pltpu.sync_copy(t_hbm.at[pl.ds(off, CHUNK)], tidx)
            c1 = pltpu.async_copy(E_hbm.at[sidx], srows, sem)
            c2 = pltpu.async_copy(R_hbm.at[ridx], rrows, sem)
            c3 = pltpu.async_copy(E_hbm.at[oidx], orows, sem)
            c4 = pltpu.async_copy(T_hbm.at[tidx], trows, sem)
            c1.wait()
            c2.wait()
            c3.wait()
            c4.wait()

            lane = lax.iota(jnp.int32, LANES)

            def allsum(v):
                for sh in (8, 4, 2, 1):
                    v = v + v.at[lane ^ sh].get(mode="promise_in_bounds")
                return v

            def group(g, _):
                eb = g * LANES
                acc = jnp.zeros((LANES,), jnp.float32)
                for j in range(LANES):
                    e = eb + j
                    x0 = (srows[e, pl.ds(0, LANES)] + rrows[e, pl.ds(0, LANES)]
                          - orows[e, pl.ds(0, LANES)])
                    x1 = (srows[e, pl.ds(16, LANES)] + rrows[e, pl.ds(16, LANES)]
                          - orows[e, pl.ds(16, LANES)])
                    x2 = (srows[e, pl.ds(32, LANES)] + rrows[e, pl.ds(32, LANES)]
                          - orows[e, pl.ds(32, LANES)])
                    x3 = (srows[e, pl.ds(48, LANES)] + rrows[e, pl.ds(48, LANES)]
                          - orows[e, pl.ds(48, LANES)])
                    t0 = trows[e, pl.ds(0, LANES)]
                    t1 = trows[e, pl.ds(16, LANES)]
                    t2 = trows[e, pl.ds(32, LANES)]
                    t3 = trows[e, pl.ds(48, LANES)]
                    p = (x0 * t0 + x1 * t1) + (x2 * t2 + x3 * t3)
                    inner = allsum(p)
                    a = (jnp.abs(x0 - t0 * inner) + jnp.abs(x1 - t1 * inner)
                         + jnp.abs(x2 - t2 * inner) + jnp.abs(x3 - t3 * inner))
                    acc = jnp.where(lane == j, allsum(a), acc)
                outv[pl.ds(eb, LANES)] = acc
                return 0

            lax.fori_loop(0, CHUNK // LANES, group, 0)
            pltpu.sync_copy(outv, out_hbm.at[pl.ds(off, CHUNK)])
            return 0

        lax.fori_loop(0, n_chunks, chunk_body, 0)

    return k(s_f, r_f, o_f, t_f, E_w, R_w, T_w)


def kernel(s, r, o, t, E_w, R_w, T_w):
    B, N = s.shape
    t_idx = t[:, :, 0].reshape(-1)
    out = _sc_score(s.reshape(-1), r.reshape(-1), o.reshape(-1), t_idx,
                    E_w, R_w, T_w)
    return out.reshape(B, N)


# speedup vs baseline: 4.0908x; 1.1867x over previous
import functools

import jax
import jax.numpy as jnp
from jax import lax
from jax.experimental import pallas as pl
from jax.experimental.pallas import tpu as pltpu
from jax.experimental.pallas import tpu_sc as plsc

DIM = 64
LANES = 16
NUM_WORKERS = 32
CHUNK = 128


def _sc_score(s_f, r_f, o_f, t_f, E_w, R_w, T_w):
    total = s_f.shape[0]
    per_w = total // NUM_WORKERS
    n_chunks = per_w // CHUNK

    mesh = plsc.VectorSubcoreMesh(core_axis_name="c", subcore_axis_name="s")

    row_buf = pltpu.VMEM((2, CHUNK, DIM), jnp.float32)

    @functools.partial(
        pl.kernel,
        mesh=mesh,
        out_type=jax.ShapeDtypeStruct((total,), jnp.float32),
        compiler_params=pltpu.CompilerParams(use_tc_tiling_on_sc=False),
        scratch_types=[
            pltpu.VMEM((per_w,), jnp.int32),
            pltpu.VMEM((per_w,), jnp.int32),
            pltpu.VMEM((per_w,), jnp.int32),
            pltpu.VMEM((per_w,), jnp.int32),
            row_buf, row_buf, row_buf, row_buf,
            pltpu.VMEM((per_w,), jnp.float32),
            pltpu.SemaphoreType.DMA,
            pltpu.SemaphoreType.DMA,
        ],
    )
    def k(s_hbm, r_hbm, o_hbm, t_hbm, E_hbm, R_hbm, T_hbm, out_hbm,
          sidx, ridx, oidx, tidx, srows, rrows, orows, trows, outv,
          sem0, sem1):
        wid = lax.axis_index("s") * 2 + lax.axis_index("c")
        base = wid * per_w

        pltpu.sync_copy(s_hbm.at[pl.ds(base, per_w)], sidx)
        pltpu.sync_copy(r_hbm.at[pl.ds(base, per_w)], ridx)
        pltpu.sync_copy(o_hbm.at[pl.ds(base, per_w)], oidx)
        pltpu.sync_copy(t_hbm.at[pl.ds(base, per_w)], tidx)

        def fire(ci, b, sem):
            sl = pl.ds(ci * CHUNK, CHUNK)
            pltpu.async_copy(E_hbm.at[sidx.at[sl]], srows.at[b], sem)
            pltpu.async_copy(R_hbm.at[ridx.at[sl]], rrows.at[b], sem)
            pltpu.async_copy(E_hbm.at[oidx.at[sl]], orows.at[b], sem)
            pltpu.async_copy(T_hbm.at[tidx.at[sl]], trows.at[b], sem)

        def drain(ci, b, sem):
            sl = pl.ds(ci * CHUNK, CHUNK)
            pltpu.make_async_copy(E_hbm.at[sidx.at[sl]], srows.at[b], sem).wait()
            pltpu.make_async_copy(R_hbm.at[ridx.at[sl]], rrows.at[b], sem).wait()
            pltpu.make_async_copy(E_hbm.at[oidx.at[sl]], orows.at[b], sem).wait()
            pltpu.make_async_copy(T_hbm.at[tidx.at[sl]], trows.at[b], sem).wait()

        lane = lax.iota(jnp.int32, LANES)

        def allsum(v):
            for sh in (8, 4, 2, 1):
                v = v + v.at[lane ^ sh].get(mode="promise_in_bounds")
            return v

        def compute(ci, b):
            sb, rb, ob, tb = srows.at[b], rrows.at[b], orows.at[b], trows.at[b]

            def group(g, _):
                eb = g * LANES
                acc = jnp.zeros((LANES,), jnp.float32)
                for j in range(LANES):
                    e = eb + j
                    x0 = (sb[e, pl.ds(0, LANES)] + rb[e, pl.ds(0, LANES)]
                          - ob[e, pl.ds(0, LANES)])
                    x1 = (sb[e, pl.ds(16, LANES)] + rb[e, pl.ds(16, LANES)]
                          - ob[e, pl.ds(16, LANES)])
                    x2 = (sb[e, pl.ds(32, LANES)] + rb[e, pl.ds(32, LANES)]
                          - ob[e, pl.ds(32, LANES)])
                    x3 = (sb[e, pl.ds(48, LANES)] + rb[e, pl.ds(48, LANES)]
                          - ob[e, pl.ds(48, LANES)])
                    t0 = tb[e, pl.ds(0, LANES)]
                    t1 = tb[e, pl.ds(16, LANES)]
                    t2 = tb[e, pl.ds(32, LANES)]
                    t3 = tb[e, pl.ds(48, LANES)]
                    p = (x0 * t0 + x1 * t1) + (x2 * t2 + x3 * t3)
                    inner = allsum(p)
                    a = (jnp.abs(x0 - t0 * inner) + jnp.abs(x1 - t1 * inner)
                         + jnp.abs(x2 - t2 * inner) + jnp.abs(x3 - t3 * inner))
                    acc = jnp.where(lane == j, allsum(a), acc)
                outv[pl.ds(ci * CHUNK + eb, LANES)] = acc
                return 0

            lax.fori_loop(0, CHUNK // LANES, group, 0)

        fire(0, 0, sem0)

        def pair(p, _):
            c0 = 2 * p
            fire(c0 + 1, 1, sem1)
            drain(c0, 0, sem0)
            compute(c0, 0)

            @pl.when(c0 + 2 < n_chunks)
            def _():
                fire(c0 + 2, 0, sem0)

            drain(c0 + 1, 1, sem1)
            compute(c0 + 1, 1)
            return 0

        lax.fori_loop(0, n_chunks // 2, pair, 0)
        pltpu.sync_copy(outv, out_hbm.at[pl.ds(base, per_w)])

    return k(s_f, r_f, o_f, t_f, E_w, R_w, T_w)


def kernel(s, r, o, t, E_w, R_w, T_w):
    B, N = s.shape
    t_idx = t[:, :, 0].reshape(-1)
    out = _sc_score(s.reshape(-1), r.reshape(-1), o.reshape(-1), t_idx,
                    E_w, R_w, T_w)
    return out.reshape(B, N)
